# Initial kernel scaffold; baseline (speedup 1.0000x reference)
import jax
import jax.numpy as jnp
from jax.experimental import pallas as pl


def kernel(src_tokens, src_lengths, embed_table, pos_table):
    raise NotImplementedError("write your pallas kernel here")


# speedup vs baseline: 1.0331x; 1.0331x over previous
import math

import jax
import jax.numpy as jnp
from jax import lax
from jax.experimental import pallas as pl
from jax.experimental.pallas import tpu as pltpu
from jax.experimental.pallas import tpu_sc as plsc

_VOCAB = 1000000
_D = 64
_PAD = 1
_B = 1024
_S = 200

_NC = 2
_NS = 16
_NW = _NC * _NS
_B_PER_W = _B // _NW

_SA = 104
_SB = _S - _SA

_SCALE = math.sqrt(_D)


def _sc_body(tok_hbm, emb_hbm, pos_hbm,
             x_hbm, mask_hbm, positions_hbm,
             tok32, mask32, pid_v, xid_a, xid_b, erows, prows, xv, sem):
  cid = lax.axis_index("c")
  sid = lax.axis_index("s")
  wid = sid * _NC + cid
  b0 = wid * _B_PER_W
  iota = lax.iota(jnp.int32, 16)

  pltpu.sync_copy(tok_hbm.at[pl.ds(b0, _B_PER_W), :], tok32)

  def row_body(r, _):
    b = b0 + r

    def chunk(c, carry):
      off = pl.multiple_of(c * 16, 16)
      t = tok32[r, pl.ds(off, 16)]
      nonpad = jnp.where(t != _PAD, 1, 0).astype(jnp.int32)
      cs = plsc.cumsum(nonpad) + carry
      pid_v[pl.ds(off, 16)] = cs * nonpad + 1
      mask32[r, pl.ds(off, 16)] = 1 - nonpad
      xs = off + iota
      xid = xs * _B + b
      pl.when(off < _SA)(lambda: plsc.store_scatter(xid_a, [xs], xid))

      def _xb():
        plsc.store_scatter(xid_b, [xs - _SA], xid)

      pl.when(off >= _SA)(_xb)
      return carry + jnp.sum(nonpad)

    carry = lax.fori_loop(0, (_S - 8) // 16, chunk, jnp.int32(0))

    t = tok32[r, pl.ds(_S - 16, 16)]
    fresh = iota >= 8
    nonpad = jnp.where(t != _PAD, 1, 0).astype(jnp.int32)
    cs = plsc.cumsum(nonpad * fresh.astype(jnp.int32)) + carry
    xs = (_S - 16) + iota
    plsc.store_scatter(pid_v, [xs], cs * nonpad + 1, mask=fresh)
    plsc.store_scatter(mask32, [jnp.full((16,), r, jnp.int32), xs],
                       1 - nonpad, mask=fresh)
    plsc.store_scatter(xid_b, [xs - _SA], xs * _B + b, mask=fresh)

    h0 = pltpu.async_copy(emb_hbm.at[tok32.at[r, pl.ds(0, _SA)]],
                          erows.at[pl.ds(0, _SA)], sem)
    h1 = pltpu.async_copy(emb_hbm.at[tok32.at[r, pl.ds(_SA, _SB)]],
                          erows.at[pl.ds(_SA, _SB)], sem)
    h2 = pltpu.async_copy(pos_hbm.at[pid_v.at[pl.ds(0, _SA)]],
                          prows.at[pl.ds(0, _SA)], sem)
    h3 = pltpu.async_copy(pos_hbm.at[pid_v.at[pl.ds(_SA, _SB)]],
                          prows.at[pl.ds(_SA, _SB)], sem)
    h0.wait()
    h1.wait()
    h2.wait()
    h3.wait()

    def madd(s, _):
      for ci in range(_D // 16):
        sl = pl.ds(ci * 16, 16)
        xv[s, sl] = erows[s, sl] * _SCALE + prows[s, sl]
      return 0

    lax.fori_loop(0, _S, madd, 0)

    pltpu.sync_copy(prows, positions_hbm.at[b])
    w0 = pltpu.async_copy(xv.at[pl.ds(0, _SA)], x_hbm.at[xid_a], sem)
    w1 = pltpu.async_copy(xv.at[pl.ds(_SA, _SB)], x_hbm.at[xid_b], sem)
    w0.wait()
    w1.wait()
    return 0

  lax.fori_loop(0, _B_PER_W, row_body, 0)
  pltpu.sync_copy(mask32, mask_hbm.at[pl.ds(b0, _B_PER_W), :])


@jax.jit
def _sc_call(src_tokens, embed_table, pos_table):
  mesh = plsc.VectorSubcoreMesh(core_axis_name="c", subcore_axis_name="s")
  out_type = (
      jax.ShapeDtypeStruct((_S * _B, _D), jnp.float32),
      jax.ShapeDtypeStruct((_B, _S), jnp.int32),
      jax.ShapeDtypeStruct((_B, _S, _D), jnp.float32),
  )
  scratch = [
      pltpu.VMEM((_B_PER_W, _S), jnp.int32),
      pltpu.VMEM((_B_PER_W, _S), jnp.int32),
      pltpu.VMEM((_S + 8,), jnp.int32),
      pltpu.VMEM((_SA,), jnp.int32),
      pltpu.VMEM((_SB,), jnp.int32),
      pltpu.VMEM((_S, _D), jnp.float32),
      pltpu.VMEM((_S, _D), jnp.float32),
      pltpu.VMEM((_S, _D), jnp.float32),
      pltpu.SemaphoreType.DMA,
  ]
  run = pl.kernel(
      _sc_body, mesh=mesh, out_type=out_type, scratch_types=scratch,
      compiler_params=pltpu.CompilerParams(
          use_tc_tiling_on_sc=False, needs_layout_passes=False))
  return run(src_tokens, embed_table, pos_table)


def kernel(src_tokens, src_lengths, embed_table, pos_table):
  del src_lengths
  x_flat, mask_i32, positions = _sc_call(
      src_tokens.astype(jnp.int32), embed_table, pos_table)
  return (x_flat.reshape(_S, _B, _D), mask_i32.astype(jnp.bool_), positions)
